# Initial kernel scaffold; baseline (speedup 1.0000x reference)
import jax
import jax.numpy as jnp
from jax.experimental import pallas as pl


def kernel(X, edge_index, edge_weight, concat_layers, H, C, W_xi, W_hi, w_ci, b_i, W_xf, W_hf, w_cf, b_f, W_xc, W_hc, b_c, W_xo, W_ho, w_co, b_o, g_h, be_h, g_c, be_c, g_o, be_o, W_fc1, b_fc1, W_fc2, b_fc2):
    raise NotImplementedError("write your pallas kernel here")


# speedup vs baseline: 7.7887x; 7.7887x over previous
import functools

import jax
import jax.numpy as jnp
from jax import lax
from jax.experimental import pallas as pl
from jax.experimental.pallas import tpu as pltpu
from jax.experimental.pallas import tpu_sc as plsc

N = 50000
N_PAD = 50048
E = 800000
HID = 64

BATCH = 2048
NBATCH = 416
E_PAD = BATCH * NBATCH
NROWS = E_PAD // 128
TILE_N = N_PAD // 16


def _run_job(tbl, out, src2d, dst2d, wf, src_v, dst_v, w_v, rows_v, acc,
             z16, sub, lo, hi, width1, gsem):
    nper = (hi - lo) // 16

    pltpu.sync_copy(z16.at[pl.ds(sub * TILE_N, TILE_N)],
                    acc.at[pl.ds(sub * TILE_N, TILE_N)])
    plsc.subcore_barrier()

    def batch_body(k, _):
        m0 = (lo + sub * nper + k) * 16
        pltpu.sync_copy(src2d.at[pl.ds(m0, 16)], src_v)
        pltpu.sync_copy(dst2d.at[pl.ds(m0, 16)], dst_v)
        pltpu.sync_copy(wf.at[pl.ds(m0 * 128, BATCH)], w_v)
        cps = [pltpu.async_copy(tbl.at[src_v.at[j]],
                                rows_v.at[pl.ds(j * 128, 128)], gsem)
               for j in range(16)]
        for cp in cps:
            cp.wait()

        def scale_body(b, _):
            wv = plsc.load_gather(w_v, [jnp.full((16,), b, jnp.int32)])
            rows_v[b, :] = rows_v[b, :] * wv
            return _
        lax.fori_loop(0, BATCH, scale_body, None)

        for j in range(16):
            pltpu.sync_copy(rows_v.at[pl.ds(j * 128, 128)],
                            acc.at[dst_v.at[j]], add=True)
        return _

    lax.fori_loop(0, nper, batch_body, None)
    plsc.subcore_barrier()
    pltpu.sync_copy(acc.at[pl.ds(sub * TILE_N, TILE_N)],
                    out.at[pl.ds(sub * TILE_N, TILE_N)])
    plsc.subcore_barrier()


def _make_scatter_pass(n_tables, jobs0, jobs1):
    n_out = 1 + max(max((j[1] for j in jobs0)), max((j[1] for j in jobs1)))

    @functools.partial(
        pl.kernel,
        out_type=[jax.ShapeDtypeStruct((N_PAD, 16), jnp.float32)] * n_out,
        mesh=plsc.VectorSubcoreMesh(core_axis_name="c", subcore_axis_name="s"),
        scratch_types=[
            pltpu.VMEM((16, 128), jnp.int32),
            pltpu.VMEM((16, 128), jnp.int32),
            pltpu.VMEM((BATCH,), jnp.float32),
            pltpu.VMEM((BATCH, 16), jnp.float32),
            pltpu.VMEM_SHARED((N_PAD, 16), jnp.float32),
            pltpu.SemaphoreType.DMA,
        ],
        compiler_params=pltpu.CompilerParams(use_tc_tiling_on_sc=False, needs_layout_passes=False),
    )
    def scatter_pass(*refs):
        tbls = refs[:n_tables]
        src2d, dst2d, wf, z16 = refs[n_tables:n_tables + 4]
        outs = refs[n_tables + 4:n_tables + 4 + n_out]
        src_v, dst_v, w_v, rows_v, acc, gsem = refs[n_tables + 4 + n_out:]
        core = lax.axis_index("c")
        sub = lax.axis_index("s")
        for cid, jobs in ((0, jobs0), (1, jobs1)):
            @pl.when(core == cid)
            def _():
                for (ti, oi, lo, hi, w1) in jobs:
                    _run_job(tbls[ti], outs[oi], src2d, dst2d, wf,
                             src_v, dst_v, w_v, rows_v, acc, z16,
                             sub, lo, hi, w1, gsem)

    return scatter_pass


def _make_deg_pass():
    @functools.partial(
        pl.kernel,
        out_type=[jax.ShapeDtypeStruct((N_PAD, 1), jnp.float32)] * 2,
        mesh=plsc.VectorSubcoreMesh(core_axis_name="c", subcore_axis_name="s"),
        scratch_types=[
            pltpu.VMEM((16, 128), jnp.int32),
            pltpu.VMEM((BATCH, 1), jnp.float32),
            pltpu.VMEM_SHARED((N_PAD, 1), jnp.float32),
        ],
        compiler_params=pltpu.CompilerParams(use_tc_tiling_on_sc=False, needs_layout_passes=False),
    )
    def deg_pass(dst2d, w1, z1, o0, o1, dst_v, w_v, acc):
        core = lax.axis_index("c")
        sub = lax.axis_index("s")
        nper = NBATCH // 32
        pltpu.sync_copy(z1.at[pl.ds(sub * TILE_N, TILE_N)],
                        acc.at[pl.ds(sub * TILE_N, TILE_N)])
        plsc.subcore_barrier()

        def batch_body(k, _):
            m0 = (core * (NBATCH // 2) + sub * nper + k) * 16
            pltpu.sync_copy(dst2d.at[pl.ds(m0, 16)], dst_v)
            pltpu.sync_copy(w1.at[pl.ds(m0 * 128, BATCH)], w_v)
            for j in range(16):
                pltpu.sync_copy(w_v.at[pl.ds(j * 128, 128)],
                                acc.at[dst_v.at[j]], add=True)
            return _

        lax.fori_loop(0, nper, batch_body, None)
        plsc.subcore_barrier()
        for cid, out in ((0, o0), (1, o1)):
            @pl.when(core == cid)
            def _():
                pltpu.sync_copy(acc.at[pl.ds(sub * TILE_N, TILE_N)],
                                out.at[pl.ds(sub * TILE_N, TILE_N)])
        plsc.subcore_barrier()

    return deg_pass


BN = 1000
GRID = N // BN


def _k1_body(va0, va1, x, c, h0, dinv_o, dinv2_o, t0, t1, t2, t3, t4):
    deg = va0[...] + va1[...] + 1.0
    dinv = lax.rsqrt(deg)
    dinv2 = dinv * dinv
    dinv_o[...] = dinv
    dinv2_o[...] = dinv2
    t0[...] = jnp.concatenate(
        [x[...] * dinv, c[...] * dinv, jnp.zeros((BN, 11), jnp.float32)],
        axis=1)
    for j, t in enumerate((t1, t2, t3, t4)):
        t[...] = h0[:, 16 * j:16 * (j + 1)] * dinv


def _k2_body(v0, v1, v2, v3, v4, x, c, h0, c0, dinv, dinv2, gx, gh, pr,
             hid_o, cell_o, out64_o, u0, u1, u2, u3, pc_o):
    dinv_ = dinv[...]
    dinv2_ = dinv2[...]
    px = dinv_ * v0[:, 0:4] + dinv2_ * x[...]
    pc = dinv_ * v0[:, 4:5] + dinv2_ * c[...]
    ph = dinv_ * jnp.concatenate([v1[...], v2[...], v3[...], v4[...]],
                                 axis=1) + dinv2_ * h0[...]
    agg = jnp.dot(px, gx[...], preferred_element_type=jnp.float32) \
        + jnp.dot(ph, gh[...], preferred_element_type=jnp.float32)
    (w_ci, w_cf, w_co, b_i, b_f, b_c, b_o,
     g_h, be_h, g_c, be_c, g_o, be_o) = [pr[j:j + 1, :] for j in range(13)]
    c0_ = c0[...]
    ig = jax.nn.sigmoid(agg[:, 0:64] + w_ci * c0_ + b_i)
    fg = jax.nn.sigmoid(agg[:, 64:128] + w_cf * c0_ + b_f)
    tg = jnp.tanh(agg[:, 128:192] + b_c)
    c1 = fg * c0_ + ig * tg
    og = jax.nn.sigmoid(agg[:, 192:256] + w_co * c1 + b_o)
    h1 = og * jnp.tanh(c1)

    def ln(v, g, b):
        mu = jnp.mean(v, axis=-1, keepdims=True)
        var = jnp.mean((v - mu) ** 2, axis=-1, keepdims=True)
        return (v - mu) * lax.rsqrt(var + 1e-5) * g + b

    hid_o[...] = ln(h1, g_h, be_h)
    cell_o[...] = ln(c1, g_c, be_c)
    out = jax.nn.relu(ln(h1, g_o, be_o))
    out64_o[...] = out
    u = out * dinv_
    for j, t in enumerate((u0, u1, u2, u3)):
        t[...] = u[:, 16 * j:16 * (j + 1)]
    pc_o[...] = pc


def _k3_body(v0, v1, v2, v3, out64, dinv, dinv2, pc, wfc1, bfc1, wfc2,
             t_o, u3p_o):
    dinv_ = dinv[...]
    pout = dinv_ * jnp.concatenate([v0[...], v1[...], v2[...], v3[...]],
                                   axis=1) + dinv2[...] * out64[...]
    z1 = jnp.dot(pout, wfc1[0:64, :], preferred_element_type=jnp.float32) \
        + pc[...] * wfc1[64:65, :] + bfc1[...]
    z1 = jax.nn.relu(z1)
    t = jnp.dot(z1, wfc2[...], preferred_element_type=jnp.float32)
    t_o[...] = t
    u3p_o[...] = jnp.concatenate(
        [t * dinv_, jnp.zeros((BN, 15), jnp.float32)], axis=1)


def _k4_body(v3a, v3b, t, x, dinv, dinv2, bfc2, out_o):
    z2 = dinv[...] * (v3a[:, 0:1] + v3b[:, 0:1]) + dinv2[...] * t[...] \
        + bfc2[...]
    out_o[...] = jnp.tanh(z2) + x[:, 0:1]


def _rowspec(width):
    return pl.BlockSpec((BN, width), lambda i: (i, 0))


def _fullspec(shape):
    return pl.BlockSpec(shape, lambda i: tuple(0 for _ in shape))


def kernel(X, edge_index, edge_weight, concat_layers, H, C,
           W_xi, W_hi, w_ci, b_i, W_xf, W_hf, w_cf, b_f,
           W_xc, W_hc, b_c, W_xo, W_ho, w_co, b_o,
           g_h, be_h, g_c, be_c, g_o, be_o,
           W_fc1, b_fc1, W_fc2, b_fc2):
    src = edge_index[0]
    dst = edge_index[1]
    w = edge_weight
    H0 = H[0]
    C0 = C[0]

    pad = E_PAD - E
    src2d = jnp.pad(src, (0, pad)).reshape(NROWS, 128)
    dst2d = jnp.pad(dst, (0, pad)).reshape(NROWS, 128)
    wf = jnp.pad(w, (0, pad))
    z16 = jnp.zeros((N_PAD, 16), jnp.float32)
    z1 = jnp.zeros((N_PAD, 1), jnp.float32)

    va0, va1 = _make_deg_pass()(dst2d, wf.reshape(E_PAD, 1), z1)

    f32 = jnp.float32
    k1 = pl.pallas_call(
        _k1_body,
        grid=(GRID,),
        in_specs=[pl.BlockSpec((BN, 1), lambda i: (i, 0))] * 2
        + [_rowspec(4), _rowspec(1), _rowspec(64)],
        out_specs=[_rowspec(1)] * 2 + [_rowspec(16)] * 5,
        out_shape=[jax.ShapeDtypeStruct((N, 1), f32)] * 2
        + [jax.ShapeDtypeStruct((N_PAD, 16), f32)] * 5,
    )
    dinv, dinv2, t0, t1, t2, t3, t4 = k1(va0, va1, X, concat_layers, H0)

    pass_b = _make_scatter_pass(
        5,
        jobs0=[(0, 0, 0, NBATCH, False), (2, 2, 0, NBATCH, False),
               (4, 4, 0, NBATCH, False)],
        jobs1=[(1, 1, 0, NBATCH, False), (3, 3, 0, NBATCH, False)])
    v0, v1, v2, v3, v4 = pass_b(t0, t1, t2, t3, t4, src2d, dst2d, wf, z16)

    gx = jnp.concatenate([W_xi, W_xf, W_xc, W_xo], axis=1)
    gh = jnp.concatenate([W_hi, W_hf, W_hc, W_ho], axis=1)
    pr = jnp.stack([w_ci, w_cf, w_co, b_i, b_f, b_c, b_o,
                    g_h, be_h, g_c, be_c, g_o, be_o])
    k2 = pl.pallas_call(
        _k2_body,
        grid=(GRID,),
        in_specs=[_rowspec(16)] * 5
        + [_rowspec(4), _rowspec(1), _rowspec(64), _rowspec(64),
           _rowspec(1), _rowspec(1),
           _fullspec((4, 256)), _fullspec((64, 256)), _fullspec((13, 64))],
        out_specs=[_rowspec(64)] * 3 + [_rowspec(16)] * 4 + [_rowspec(1)],
        out_shape=[jax.ShapeDtypeStruct((N, 64), f32)] * 3
        + [jax.ShapeDtypeStruct((N_PAD, 16), f32)] * 4
        + [jax.ShapeDtypeStruct((N, 1), f32)],
    )
    hid, cell, out64, u0, u1, u2, u3t, pc = k2(
        v0, v1, v2, v3, v4, X, concat_layers, H0, C0, dinv, dinv2,
        gx, gh, pr)

    pass_c = _make_scatter_pass(
        4,
        jobs0=[(0, 0, 0, NBATCH, False), (2, 2, 0, NBATCH, False)],
        jobs1=[(1, 1, 0, NBATCH, False), (3, 3, 0, NBATCH, False)])
    vc0, vc1, vc2, vc3 = pass_c(u0, u1, u2, u3t, src2d, dst2d, wf, z16)

    k3 = pl.pallas_call(
        _k3_body,
        grid=(GRID,),
        in_specs=[_rowspec(16)] * 4
        + [_rowspec(64), _rowspec(1), _rowspec(1), _rowspec(1),
           _fullspec((65, 64)), _fullspec((1, 64)), _fullspec((64, 1))],
        out_specs=[_rowspec(1), _rowspec(16)],
        out_shape=[jax.ShapeDtypeStruct((N, 1), f32),
                   jax.ShapeDtypeStruct((N_PAD, 16), f32)],
    )
    t_fc2, u3p = k3(vc0, vc1, vc2, vc3, out64, dinv, dinv2, pc,
                    W_fc1, b_fc1.reshape(1, 64), W_fc2)

    pass_d = _make_scatter_pass(
        1,
        jobs0=[(0, 0, 0, NBATCH // 2, False)],
        jobs1=[(0, 1, NBATCH // 2, NBATCH, False)])
    vd0, vd1 = pass_d(u3p, src2d, dst2d, wf, z16)

    k4 = pl.pallas_call(
        _k4_body,
        grid=(GRID,),
        in_specs=[_rowspec(16)] * 2
        + [_rowspec(1), _rowspec(4), _rowspec(1), _rowspec(1),
           _fullspec((1, 1))],
        out_specs=_rowspec(1),
        out_shape=jax.ShapeDtypeStruct((N, 1), f32),
    )
    out = k4(vd0, vd1, t_fc2, X, dinv, dinv2, b_fc2.reshape(1, 1))

    return (out, hid[None], cell[None])
